# baseline (speedup 1.0000x reference)
import functools

import jax
import jax.numpy as jnp
from jax import lax
from jax.experimental import pallas as pl
from jax.experimental.pallas import tpu as pltpu
from jax.experimental.pallas import tpu_sc as plsc

_N = 10000
_E = 160000
_D = 256
_C = 40

_HALF = 5120
_NPAD = 2 * _HALF
_EPT = _E // 16
_K = 64
_SUB = _EPT
_SEL = _SUB + 4 * _K
_AGG_ROWS = 2 * _HALF + 256
_ZROWS = _AGG_ROWS // 16
_HROWS = 40


def _sc_body(x_hbm, src_hbm, dst_hbm, agg_hbm, deg_hbm,
             sel_src, sel_dst, cs0, cd0, idx48,
             rows0, hist, agg_sp, deg_sp, sem0):
    c = lax.axis_index("c")
    s = lax.axis_index("s")
    lo = c * _HALF

    zero16f = jnp.zeros((16,), jnp.float32)
    ones16f = jnp.ones((16,), jnp.float32)
    zero16i = jnp.zeros((16,), jnp.int32)
    iota16 = lax.iota(jnp.int32, 16)

    def _zh(i, carry):
        for j in range(128 // 16):
            hist[i, pl.ds(j * 16, 16)] = zero16f
        return carry
    lax.fori_loop(0, _HROWS, _zh, 0)

    @pl.when(s == 0)
    def _():
        pltpu.sync_copy(hist, deg_sp)

    def _zr(i, carry):
        for j in range(128 // 16):
            rows0[i, pl.ds(j * 16, 16)] = zero16f
        return carry
    lax.fori_loop(0, 2 * _K, _zr, 0)
    r0 = s * _ZROWS
    zero_dmas = [
        pltpu.async_copy(rows0, agg_sp.at[pl.ds(r0 + k * 2 * _K, 2 * _K)],
                         sem0)
        for k in range(_ZROWS // (2 * _K))
    ]
    zero_dmas.append(
        pltpu.async_copy(rows0.at[pl.ds(0, _ZROWS % (2 * _K))],
                         agg_sp.at[pl.ds(r0 + (_ZROWS // (2 * _K)) * 2 * _K,
                                         _ZROWS % (2 * _K))], sem0))

    e0 = s * _EPT
    dum_dst = jnp.full((16,), _HALF, jnp.int32)

    def _build(k, cs, cd):
        base = k * _K
        for j in range(_K // 16):
            sv = sel_src[pl.ds(base + j * 16, 16)]
            dv = sel_dst[pl.ds(base + j * 16, 16)]
            ii = iota16 * 2 + j * 32
            plsc.store_scatter(cs, [ii], sv * 2)
            plsc.store_scatter(cs, [ii + 1], sv * 2 + 1)
            plsc.store_scatter(cd, [ii], dv * 2)
            plsc.store_scatter(cd, [ii + 1], dv * 2 + 1)

    def _process(off, size):
        pltpu.sync_copy(src_hbm.at[pl.ds(e0 + off, size)],
                        sel_src.at[pl.ds(0, size)])
        pltpu.sync_copy(dst_hbm.at[pl.ds(e0 + off, size)],
                        sel_dst.at[pl.ds(0, size)])

        def _compact(g, cnt):
            dvec = sel_dst[pl.ds(g * 16, 16)]
            svec = sel_src[pl.ds(g * 16, 16)]
            local = dvec - lo
            mask = (local >= 0) & (local < _HALF)
            localc = jnp.clip(local, 0, _HALF - 1)
            plsc.addupdate_scatter(
                hist,
                [jnp.right_shift(localc, 7), jnp.bitwise_and(localc, 127)],
                ones16f, mask=mask)
            plsc.store_compressed(sel_dst.at[pl.ds(cnt, 16)], local, mask=mask)
            plsc.store_compressed(sel_src.at[pl.ds(cnt, 16)], svec, mask=mask)
            return cnt + jnp.sum(jnp.where(mask, 1, 0))
        cnt = lax.fori_loop(0, size // 16, _compact, jnp.int32(0))

        for j in range(4 * _K // 16):
            sel_src[pl.ds(cnt + j * 16, 16)] = zero16i
            sel_dst[pl.ds(cnt + j * 16, 16)] = dum_dst

        for d in zero_dmas:
            d.wait()
        plsc.subcore_barrier()

        nch = jnp.right_shift(cnt + (_K - 1), 6)

        def _chunk(k, carry):
            _build(k, cs0, cd0)
            pltpu.async_copy(x_hbm.at[cs0], rows0, sem0).wait()
            pltpu.sync_copy(rows0, agg_sp.at[cd0], add=True)
            return carry
        lax.fori_loop(0, nch, _chunk, 0)

    _process(0, _EPT)

    for j in range(_HROWS // 16):
        idx48[pl.ds(j * 16, 16)] = iota16 + j * 16
    idx48[pl.ds(_HROWS - 16, 16)] = iota16 + (_HROWS - 16)
    pltpu.sync_copy(hist, deg_sp.at[idx48], add=True)

    plsc.subcore_barrier()

    a0 = s * (2 * _HALF // 16)
    pltpu.sync_copy(agg_sp.at[pl.ds(a0, 2 * _HALF // 16)],
                    agg_hbm.at[pl.ds(2 * lo + a0, 2 * _HALF // 16)])

    @pl.when(s == 0)
    def _():
        pltpu.sync_copy(deg_sp, deg_hbm.at[pl.ds(c * _HROWS, _HROWS)])


def _sc_stage(x2, src, dst):
    mesh = plsc.VectorSubcoreMesh(core_axis_name="c", subcore_axis_name="s")
    return pl.kernel(
        _sc_body,
        out_type=[
            jax.ShapeDtypeStruct((2 * _NPAD, 128), jnp.float32),
            jax.ShapeDtypeStruct((2 * _HROWS, 128), jnp.float32),
        ],
        mesh=mesh,
        compiler_params=pltpu.CompilerParams(needs_layout_passes=False),
        scratch_types=[
            pltpu.VMEM((_SEL,), jnp.int32),
            pltpu.VMEM((_SEL,), jnp.int32),
            pltpu.VMEM((2 * _K,), jnp.int32),
            pltpu.VMEM((2 * _K,), jnp.int32),
            pltpu.VMEM((_HROWS,), jnp.int32),
            pltpu.VMEM((2 * _K, 128), jnp.float32),
            pltpu.VMEM((_HROWS, 128), jnp.float32),
            pltpu.VMEM_SHARED((_AGG_ROWS, 128), jnp.float32),
            pltpu.VMEM_SHARED((_HROWS, 128), jnp.float32),
            pltpu.SemaphoreType.DMA,
        ],
    )(x2, src, dst)


_BLK = 2048


def _xr_body(x_ref, wr_ref, xr_ref):
    xr_ref[:] = jnp.dot(x_ref[:], wr_ref[:], preferred_element_type=jnp.float32)


def _xr_stage(x, W_r):
    nblk = pl.cdiv(_N, _BLK)
    return pl.pallas_call(
        _xr_body,
        grid=(nblk,),
        in_specs=[
            pl.BlockSpec((_BLK, _D), lambda i: (i, 0)),
            pl.BlockSpec((_D, _D), lambda i: (0, 0)),
        ],
        out_specs=pl.BlockSpec((_BLK, _D), lambda i: (i, 0)),
        out_shape=jax.ShapeDtypeStruct((_N, _D), jnp.float32),
    )(x, W_r)


def _dense_body(agg_ref, deg_ref, xr_ref, bl_ref, wl_ref, wn_ref,
                out_ref, h_ref):
    a = agg_ref[:].reshape(_BLK, _D) / jnp.maximum(deg_ref[:], 1.0)
    h = (jnp.dot(a, wl_ref[:], preferred_element_type=jnp.float32)
         + bl_ref[:]
         + xr_ref[:])
    h_ref[:] = h
    rnorm = jnp.sqrt(jnp.sum(h * h, axis=1, keepdims=True))
    hn = h / jnp.maximum(rnorm, 1e-12)
    wn = wn_ref[:]
    cnorm = jnp.sqrt(jnp.sum(wn * wn, axis=0, keepdims=True))
    wn = wn / jnp.maximum(cnorm, 1e-12)
    out_ref[:] = 10.0 * jnp.dot(hn, wn, preferred_element_type=jnp.float32)


def _dense_stage(agg, deg, xr, W_l, b_l, W_n):
    nblk = pl.cdiv(_N, _BLK)
    return pl.pallas_call(
        _dense_body,
        grid=(nblk,),
        in_specs=[
            pl.BlockSpec((2 * _BLK, 128), lambda i: (i, 0)),
            pl.BlockSpec((_BLK, 1), lambda i: (i, 0)),
            pl.BlockSpec((_BLK, _D), lambda i: (i, 0)),
            pl.BlockSpec((1, _D), lambda i: (0, 0)),
            pl.BlockSpec((_D, _D), lambda i: (0, 0)),
            pl.BlockSpec((_D, _C), lambda i: (0, 0)),
        ],
        out_specs=[
            pl.BlockSpec((_BLK, _C), lambda i: (i, 0)),
            pl.BlockSpec((_BLK, _D), lambda i: (i, 0)),
        ],
        out_shape=[
            jax.ShapeDtypeStruct((_N, _C), jnp.float32),
            jax.ShapeDtypeStruct((_N, _D), jnp.float32),
        ],
    )(agg, deg, xr, b_l, W_l, W_n)


def kernel(x, edge_index, W_l, b_l, W_r, W_n):
    src = edge_index[0]
    dst = edge_index[1]
    agg_2d, deg_2d = _sc_stage(x.reshape(2 * _N, 128), src, dst)
    deg = deg_2d.reshape(_NPAD, 1)
    xr = _xr_stage(x, W_r)
    out, h = _dense_stage(agg_2d, deg, xr, W_l, b_l.reshape(1, _D), W_n)
    return (out, x, h)


# speedup vs baseline: 1.0006x; 1.0006x over previous
import functools

import jax
import jax.numpy as jnp
from jax import lax
from jax.experimental import pallas as pl
from jax.experimental.pallas import tpu as pltpu
from jax.experimental.pallas import tpu_sc as plsc

_N = 10000
_E = 160000
_D = 256
_C = 40

_HALF = 5120
_NPAD = 2 * _HALF
_EPT = _E // 16
_K = 64
_SUB = _EPT
_SEL = _SUB + 4 * _K
_AGG_ROWS = 2 * _HALF + 256
_ZROWS = _AGG_ROWS // 16
_HROWS = 40


def _sc_body(x_hbm, src_hbm, dst_hbm, agg_hbm, deg_hbm,
             sel_src, sel_dst, cs0, cd0, idx48,
             rows0, hist, agg_sp, deg_sp, sem0):
    c = lax.axis_index("c")
    s = lax.axis_index("s")
    lo = c * _HALF

    zero16f = jnp.zeros((16,), jnp.float32)
    ones16f = jnp.ones((16,), jnp.float32)
    zero16i = jnp.zeros((16,), jnp.int32)
    iota16 = lax.iota(jnp.int32, 16)

    def _zh(i, carry):
        for j in range(128 // 16):
            hist[i, pl.ds(j * 16, 16)] = zero16f
        return carry
    lax.fori_loop(0, _HROWS, _zh, 0)

    @pl.when(s == 0)
    def _():
        pltpu.sync_copy(hist, deg_sp)

    def _zr(i, carry):
        for j in range(128 // 16):
            rows0[i, pl.ds(j * 16, 16)] = zero16f
        return carry
    lax.fori_loop(0, 2 * _K, _zr, 0)
    r0 = s * _ZROWS
    zero_dmas = [
        pltpu.async_copy(rows0, agg_sp.at[pl.ds(r0 + k * 2 * _K, 2 * _K)],
                         sem0)
        for k in range(_ZROWS // (2 * _K))
    ]
    zero_dmas.append(
        pltpu.async_copy(rows0.at[pl.ds(0, _ZROWS % (2 * _K))],
                         agg_sp.at[pl.ds(r0 + (_ZROWS // (2 * _K)) * 2 * _K,
                                         _ZROWS % (2 * _K))], sem0))

    e0 = s * _EPT
    dum_dst = jnp.full((16,), _HALF, jnp.int32)

    def _build(k, cs, cd):
        base = k * _K
        for j in range(_K // 16):
            sv = sel_src[pl.ds(base + j * 16, 16)]
            dv = sel_dst[pl.ds(base + j * 16, 16)]
            ii = iota16 * 2 + j * 32
            plsc.store_scatter(cs, [ii], sv * 2)
            plsc.store_scatter(cs, [ii + 1], sv * 2 + 1)
            plsc.store_scatter(cd, [ii], dv * 2)
            plsc.store_scatter(cd, [ii + 1], dv * 2 + 1)

    def _process(off, size):
        pltpu.sync_copy(src_hbm.at[pl.ds(e0 + off, size)],
                        sel_src.at[pl.ds(0, size)])
        pltpu.sync_copy(dst_hbm.at[pl.ds(e0 + off, size)],
                        sel_dst.at[pl.ds(0, size)])

        def _compact(g, cnt):
            dvec = sel_dst[pl.ds(g * 16, 16)]
            svec = sel_src[pl.ds(g * 16, 16)]
            local = dvec - lo
            mask = (local >= 0) & (local < _HALF)
            localc = jnp.clip(local, 0, _HALF - 1)
            plsc.addupdate_scatter(
                hist,
                [jnp.right_shift(localc, 7), jnp.bitwise_and(localc, 127)],
                ones16f, mask=mask)
            plsc.store_compressed(sel_dst.at[pl.ds(cnt, 16)], local, mask=mask)
            plsc.store_compressed(sel_src.at[pl.ds(cnt, 16)], svec, mask=mask)
            return cnt + jnp.sum(jnp.where(mask, 1, 0))
        cnt = lax.fori_loop(0, size // 16, _compact, jnp.int32(0))

        for j in range(4 * _K // 16):
            sel_src[pl.ds(cnt + j * 16, 16)] = zero16i
            sel_dst[pl.ds(cnt + j * 16, 16)] = dum_dst

        for d in zero_dmas:
            d.wait()
        plsc.subcore_barrier()

        nch = jnp.right_shift(cnt + (_K - 1), 6)

        def _chunk(k, carry):
            _build(k, cs0, cd0)
            pltpu.async_copy(x_hbm.at[cs0], rows0, sem0).wait()
            pltpu.sync_copy(rows0, agg_sp.at[cd0], add=True)
            return carry
        lax.fori_loop(0, nch, _chunk, 0)

    _process(0, _EPT)

    for j in range(_HROWS // 16):
        idx48[pl.ds(j * 16, 16)] = iota16 + j * 16
    idx48[pl.ds(_HROWS - 16, 16)] = iota16 + (_HROWS - 16)
    pltpu.sync_copy(hist, deg_sp.at[idx48], add=True)

    plsc.subcore_barrier()

    a0 = s * (2 * _HALF // 16)
    pltpu.sync_copy(agg_sp.at[pl.ds(a0, 2 * _HALF // 16)],
                    agg_hbm.at[pl.ds(2 * lo + a0, 2 * _HALF // 16)])

    @pl.when(s == 0)
    def _():
        pltpu.sync_copy(deg_sp, deg_hbm.at[pl.ds(c * _HROWS, _HROWS)])


def _sc_stage(x2, src, dst):
    mesh = plsc.VectorSubcoreMesh(core_axis_name="c", subcore_axis_name="s")
    return pl.kernel(
        _sc_body,
        out_type=[
            jax.ShapeDtypeStruct((2 * _NPAD, 128), jnp.float32),
            jax.ShapeDtypeStruct((2 * _HROWS, 128), jnp.float32),
        ],
        mesh=mesh,
        compiler_params=pltpu.CompilerParams(needs_layout_passes=False),
        scratch_types=[
            pltpu.VMEM((_SEL,), jnp.int32),
            pltpu.VMEM((_SEL,), jnp.int32),
            pltpu.VMEM((2 * _K,), jnp.int32),
            pltpu.VMEM((2 * _K,), jnp.int32),
            pltpu.VMEM((_HROWS,), jnp.int32),
            pltpu.VMEM((2 * _K, 128), jnp.float32),
            pltpu.VMEM((_HROWS, 128), jnp.float32),
            pltpu.VMEM_SHARED((_AGG_ROWS, 128), jnp.float32),
            pltpu.VMEM_SHARED((_HROWS, 128), jnp.float32),
            pltpu.SemaphoreType.DMA,
        ],
    )(x2, src, dst)


_BLK = 1024


def _xr_body(x_ref, wr_ref, xr_ref):
    xr_ref[:] = jnp.dot(x_ref[:], wr_ref[:], preferred_element_type=jnp.float32)


def _xr_stage(x, W_r):
    nblk = pl.cdiv(_N, _BLK)
    return pl.pallas_call(
        _xr_body,
        grid=(nblk,),
        in_specs=[
            pl.BlockSpec((_BLK, _D), lambda i: (i, 0)),
            pl.BlockSpec((_D, _D), lambda i: (0, 0)),
        ],
        out_specs=pl.BlockSpec((_BLK, _D), lambda i: (i, 0)),
        out_shape=jax.ShapeDtypeStruct((_N, _D), jnp.float32),
    )(x, W_r)


def _dense_body(agg_ref, deg_ref, xr_ref, bl_ref, wl_ref, wn_ref,
                out_ref, h_ref):
    a = agg_ref[:].reshape(_BLK, _D) / jnp.maximum(deg_ref[:], 1.0)
    h = (jnp.dot(a, wl_ref[:], preferred_element_type=jnp.float32)
         + bl_ref[:]
         + xr_ref[:])
    h_ref[:] = h
    rnorm = jnp.sqrt(jnp.sum(h * h, axis=1, keepdims=True))
    hn = h / jnp.maximum(rnorm, 1e-12)
    wn = wn_ref[:]
    cnorm = jnp.sqrt(jnp.sum(wn * wn, axis=0, keepdims=True))
    wn = wn / jnp.maximum(cnorm, 1e-12)
    out_ref[:] = 10.0 * jnp.dot(hn, wn, preferred_element_type=jnp.float32)


def _dense_stage(agg, deg, xr, W_l, b_l, W_n):
    nblk = pl.cdiv(_N, _BLK)
    return pl.pallas_call(
        _dense_body,
        grid=(nblk,),
        in_specs=[
            pl.BlockSpec((2 * _BLK, 128), lambda i: (i, 0)),
            pl.BlockSpec((_BLK, 1), lambda i: (i, 0)),
            pl.BlockSpec((_BLK, _D), lambda i: (i, 0)),
            pl.BlockSpec((1, _D), lambda i: (0, 0)),
            pl.BlockSpec((_D, _D), lambda i: (0, 0)),
            pl.BlockSpec((_D, _C), lambda i: (0, 0)),
        ],
        out_specs=[
            pl.BlockSpec((_BLK, _C), lambda i: (i, 0)),
            pl.BlockSpec((_BLK, _D), lambda i: (i, 0)),
        ],
        out_shape=[
            jax.ShapeDtypeStruct((_N, _C), jnp.float32),
            jax.ShapeDtypeStruct((_N, _D), jnp.float32),
        ],
    )(agg, deg, xr, b_l, W_l, W_n)


def kernel(x, edge_index, W_l, b_l, W_r, W_n):
    src = edge_index[0]
    dst = edge_index[1]
    agg_2d, deg_2d = _sc_stage(x.reshape(2 * _N, 128), src, dst)
    deg = deg_2d.reshape(_NPAD, 1)
    xr = _xr_stage(x, W_r)
    out, h = _dense_stage(agg_2d, deg, xr, W_l, b_l.reshape(1, _D), W_n)
    return (out, x, h)
